# baseline (speedup 1.0000x reference)
import jax
import jax.numpy as jnp
from jax.experimental import pallas as pl
from jax.experimental.pallas import tpu as pltpu

_N = 4096
_L = 65536
_BLK = 2048
_STEPS = _L // _BLK
_IDX_MASK = _BLK - 1
_VAL_MASK = ~_IDX_MASK


def _scan_body(pp_ref, pn_ref, gtTs_ref, gt_ref, out_ref, run_min, run_vals):
    i = pl.program_id(0)

    @pl.when(i == 0)
    def _init():
        run_min[...] = jnp.full((_N, 1), jnp.iinfo(jnp.int32).max, jnp.int32)
        run_vals[...] = jnp.zeros((_N, 6), jnp.float32)

    p = pp_ref[...]
    gtTs = gtTs_ref[...]
    dotn = jax.lax.dot_general(p, gtTs, (((1,), (0,)), ((), ())),
                               preferred_element_type=jnp.float32)
    g2 = 0.25 * jnp.sum(gtTs * gtTs, axis=0, keepdims=True)
    p2 = jnp.sum(p * p, axis=1, keepdims=True)
    d2 = jnp.maximum((p2 + g2) + dotn, 0.0)
    bits = jax.lax.bitcast_convert_type(d2, jnp.int32)
    iota = jax.lax.broadcasted_iota(jnp.int32, (_N, _BLK), 1)
    packed = (bits & _VAL_MASK) | iota
    pmin = jnp.min(packed, axis=1, keepdims=True)
    j = pmin & _IDX_MASK
    onehot = (iota == j).astype(jnp.float32)
    vals = jax.lax.dot_general(onehot, gt_ref[...], (((1,), (0,)), ((), ())),
                               preferred_element_type=jnp.float32)
    mkey = pmin & _VAL_MASK
    upd = mkey < run_min[...]
    run_min[...] = jnp.where(upd, mkey, run_min[...])
    run_vals[...] = jnp.where(upd, vals, run_vals[...])

    @pl.when(i == _STEPS - 1)
    def _fin():
        pts = run_vals[:, 0:3]
        nrm = run_vals[:, 3:6]
        pp = pp_ref[...]
        inlier = jnp.mean((pp - pts) ** 2)
        pn = pn_ref[...]
        pnu = pn / jnp.maximum(
            jnp.sqrt(jnp.sum(pn * pn, axis=1, keepdims=True)), 1e-4)
        gnu = nrm / jnp.maximum(
            jnp.sqrt(jnp.sum(nrm * nrm, axis=1, keepdims=True)), 1e-4)
        cos = jnp.sum(pnu * gnu, axis=1, keepdims=True)
        out_ref[...] = jnp.reshape(inlier + jnp.mean(1.0 - cos), (1, 1))


def kernel(pred_feat, pred_decoder, input_data, gt_data):
    del pred_decoder, input_data
    pred_pts = pred_feat[:, :3]
    pred_nrm = pred_feat[:, 3:]
    gauss = jax.random.normal(jax.random.key(1), pred_pts.shape,
                              dtype=pred_pts.dtype)
    pred_pts = jnp.where(jnp.any(jnp.isnan(pred_pts)), gauss, pred_pts)

    gtTs = -2.0 * gt_data[:, :3].T

    out = pl.pallas_call(
        _scan_body,
        grid=(_STEPS,),
        in_specs=[
            pl.BlockSpec((_N, 3), lambda i: (0, 0)),
            pl.BlockSpec((_N, 3), lambda i: (0, 0)),
            pl.BlockSpec((3, _BLK), lambda i: (0, i)),
            pl.BlockSpec((_BLK, 6), lambda i: (i, 0)),
        ],
        out_specs=pl.BlockSpec((1, 1), lambda i: (0, 0)),
        out_shape=jax.ShapeDtypeStruct((1, 1), jnp.float32),
        scratch_shapes=[
            pltpu.VMEM((_N, 1), jnp.int32),
            pltpu.VMEM((_N, 6), jnp.float32),
        ],
    )(pred_pts, pred_nrm, gtTs, gt_data)
    return out.reshape(())


# speedup vs baseline: 1.0713x; 1.0713x over previous
import jax
import jax.numpy as jnp
from jax.experimental import pallas as pl
from jax.experimental.pallas import tpu as pltpu

_N = 4096
_L = 65536
_BLK = 1024
_STEPS = _L // _BLK
_IDX_MASK = _BLK - 1
_VAL_MASK = ~_IDX_MASK


def _scan_body(pp_ref, pn_ref, gtTs_ref, gt_ref, out_ref, run_min, run_vals):
    i = pl.program_id(0)

    @pl.when(i == 0)
    def _init():
        run_min[...] = jnp.full((_N, 1), jnp.iinfo(jnp.int32).max, jnp.int32)
        run_vals[...] = jnp.zeros((_N, 6), jnp.float32)

    p = pp_ref[...]
    gtTs = gtTs_ref[...]
    dotn = jax.lax.dot_general(p, gtTs, (((1,), (0,)), ((), ())),
                               preferred_element_type=jnp.float32)
    g2 = 0.25 * jnp.sum(gtTs * gtTs, axis=0, keepdims=True)
    p2 = jnp.sum(p * p, axis=1, keepdims=True)
    d2 = jnp.maximum((p2 + g2) + dotn, 0.0)
    bits = jax.lax.bitcast_convert_type(d2, jnp.int32)
    iota = jax.lax.broadcasted_iota(jnp.int32, (_N, _BLK), 1)
    packed = (bits & _VAL_MASK) | iota
    pmin = jnp.min(packed, axis=1, keepdims=True)
    j = pmin & _IDX_MASK
    onehot = (iota == j).astype(jnp.float32)
    vals = jax.lax.dot_general(onehot, gt_ref[...], (((1,), (0,)), ((), ())),
                               preferred_element_type=jnp.float32)
    mkey = pmin & _VAL_MASK
    upd = mkey < run_min[...]
    run_min[...] = jnp.where(upd, mkey, run_min[...])
    run_vals[...] = jnp.where(upd, vals, run_vals[...])

    @pl.when(i == _STEPS - 1)
    def _fin():
        pts = run_vals[:, 0:3]
        nrm = run_vals[:, 3:6]
        pp = pp_ref[...]
        inlier = jnp.mean((pp - pts) ** 2)
        pn = pn_ref[...]
        pnu = pn / jnp.maximum(
            jnp.sqrt(jnp.sum(pn * pn, axis=1, keepdims=True)), 1e-4)
        gnu = nrm / jnp.maximum(
            jnp.sqrt(jnp.sum(nrm * nrm, axis=1, keepdims=True)), 1e-4)
        cos = jnp.sum(pnu * gnu, axis=1, keepdims=True)
        out_ref[...] = jnp.reshape(inlier + jnp.mean(1.0 - cos), (1, 1))


def kernel(pred_feat, pred_decoder, input_data, gt_data):
    del pred_decoder, input_data
    pred_pts = pred_feat[:, :3]
    pred_nrm = pred_feat[:, 3:]
    gauss = jax.random.normal(jax.random.key(1), pred_pts.shape,
                              dtype=pred_pts.dtype)
    pred_pts = jnp.where(jnp.any(jnp.isnan(pred_pts)), gauss, pred_pts)

    gtTs = -2.0 * gt_data[:, :3].T

    out = pl.pallas_call(
        _scan_body,
        grid=(_STEPS,),
        in_specs=[
            pl.BlockSpec((_N, 3), lambda i: (0, 0)),
            pl.BlockSpec((_N, 3), lambda i: (0, 0)),
            pl.BlockSpec((3, _BLK), lambda i: (0, i)),
            pl.BlockSpec((_BLK, 6), lambda i: (i, 0)),
        ],
        out_specs=pl.BlockSpec((1, 1), lambda i: (0, 0)),
        out_shape=jax.ShapeDtypeStruct((1, 1), jnp.float32),
        scratch_shapes=[
            pltpu.VMEM((_N, 1), jnp.int32),
            pltpu.VMEM((_N, 6), jnp.float32),
        ],
    )(pred_pts, pred_nrm, gtTs, gt_data)
    return out.reshape(())


# speedup vs baseline: 1.1184x; 1.0440x over previous
import jax
import jax.numpy as jnp
from jax.experimental import pallas as pl
from jax.experimental.pallas import tpu as pltpu

_N = 4096
_L = 65536
_BLK = 1024
_STEPS = _L // _BLK
_IDX_MASK = _BLK - 1
_VAL_MASK = ~_IDX_MASK


def _scan_body(pp_ref, pn_ref, gtTs_ref, gt_ref, out_ref, run_min, run_vals):
    i = pl.program_id(0)

    @pl.when(i == 0)
    def _init():
        run_min[...] = jnp.full((_N, 1), jnp.iinfo(jnp.int32).max, jnp.int32)
        run_vals[...] = jnp.zeros((_N, 6), jnp.float32)

    p = pp_ref[...]
    gtTs = gtTs_ref[...]
    dotn = jax.lax.dot_general(p, gtTs, (((1,), (0,)), ((), ())),
                               preferred_element_type=jnp.float32)
    g2 = 0.25 * jnp.sum(gtTs * gtTs, axis=0, keepdims=True)
    p2 = jnp.sum(p * p, axis=1, keepdims=True)
    d2 = (p2 + g2) + dotn
    bits = jax.lax.bitcast_convert_type(d2, jnp.int32)
    iota = jax.lax.broadcasted_iota(jnp.int32, (_N, _BLK), 1)
    packed = (bits & _VAL_MASK) | iota
    pmin = jnp.min(packed, axis=1, keepdims=True)
    j = pmin & _IDX_MASK
    onehot = (iota == j).astype(jnp.float32)
    vals = jax.lax.dot_general(onehot, gt_ref[...], (((1,), (0,)), ((), ())),
                               preferred_element_type=jnp.float32)
    mkey = pmin & _VAL_MASK
    upd = mkey < run_min[...]
    run_min[...] = jnp.where(upd, mkey, run_min[...])
    run_vals[...] = jnp.where(upd, vals, run_vals[...])

    @pl.when(i == _STEPS - 1)
    def _fin():
        pts = run_vals[:, 0:3]
        nrm = run_vals[:, 3:6]
        pp = pp_ref[...]
        inlier = jnp.mean((pp - pts) ** 2)
        pn = pn_ref[...]
        pnu = pn / jnp.maximum(
            jnp.sqrt(jnp.sum(pn * pn, axis=1, keepdims=True)), 1e-4)
        gnu = nrm / jnp.maximum(
            jnp.sqrt(jnp.sum(nrm * nrm, axis=1, keepdims=True)), 1e-4)
        cos = jnp.sum(pnu * gnu, axis=1, keepdims=True)
        out_ref[...] = jnp.reshape(inlier + jnp.mean(1.0 - cos), (1, 1))


def kernel(pred_feat, pred_decoder, input_data, gt_data):
    del pred_decoder, input_data
    pred_pts = pred_feat[:, :3]
    pred_nrm = pred_feat[:, 3:]
    gauss = jax.random.normal(jax.random.key(1), pred_pts.shape,
                              dtype=pred_pts.dtype)
    pred_pts = jnp.where(jnp.any(jnp.isnan(pred_pts)), gauss, pred_pts)

    gtTs = -2.0 * gt_data[:, :3].T

    out = pl.pallas_call(
        _scan_body,
        grid=(_STEPS,),
        in_specs=[
            pl.BlockSpec((_N, 3), lambda i: (0, 0)),
            pl.BlockSpec((_N, 3), lambda i: (0, 0)),
            pl.BlockSpec((3, _BLK), lambda i: (0, i)),
            pl.BlockSpec((_BLK, 6), lambda i: (i, 0)),
        ],
        out_specs=pl.BlockSpec((1, 1), lambda i: (0, 0)),
        out_shape=jax.ShapeDtypeStruct((1, 1), jnp.float32),
        scratch_shapes=[
            pltpu.VMEM((_N, 1), jnp.int32),
            pltpu.VMEM((_N, 6), jnp.float32),
        ],
    )(pred_pts, pred_nrm, gtTs, gt_data)
    return out.reshape(())
